# baseline (speedup 1.0000x reference)
import functools

import jax
import jax.numpy as jnp
from jax import lax
from jax.experimental import pallas as pl
from jax.experimental.pallas import tpu as pltpu
from jax.experimental.pallas import tpu_sc as plsc

_INV_STEPS = 1.0 / 16.0
_NBUF = 3


@functools.cache
def _sc_kernel(steps, rows, cols):
    info = plsc.get_sparse_core_info()
    nc, ns, lanes = info.num_cores, info.num_subcores, info.num_lanes
    nw = nc * ns
    rw = rows // nw
    ch = rw // 2
    nch = rw // ch
    nt = steps * nch
    ndh = ch // 2
    groups = ndh * cols // lanes
    mesh = plsc.VectorSubcoreMesh(core_axis_name="c", subcore_axis_name="s")

    @functools.partial(
        pl.kernel,
        out_type=jax.ShapeDtypeStruct((steps, rows, cols), jnp.float32),
        mesh=mesh,
        scratch_types=[
            pltpu.VMEM((lanes,), jnp.int32),
            [pltpu.VMEM((ch, cols), jnp.float32) for _ in range(_NBUF)],
            pltpu.VMEM((ndh, cols), jnp.float32),
            [pltpu.SemaphoreType.DMA for _ in range(_NBUF)],
            [pltpu.SemaphoreType.DMA for _ in range(_NBUF)],
        ],
    )
    def k(iv_hbm, d_hbm, nd_hbm, o_hbm, iv_v, bufs, ndbuf, in_sems, out_sems):
        wid = lax.axis_index("s") * nc + lax.axis_index("c")
        base = wid * rw
        pltpu.sync_copy(iv_hbm, iv_v)
        it = iv_v[...][0]

        def chunk_coords(t):
            return t // nch, base + (t % nch) * ch

        def start_in(t, b):
            s, lo = chunk_coords(t)
            pltpu.make_async_copy(
                d_hbm.at[s, pl.ds(lo, ch)], bufs[b], in_sems[b]
            ).start()

        def wait_in(b):
            pltpu.make_async_copy(
                d_hbm.at[0, pl.ds(base, ch)], bufs[b], in_sems[b]
            ).wait()

        def start_out(t, b):
            s, lo = chunk_coords(t)
            pltpu.make_async_copy(
                bufs[b], o_hbm.at[s, pl.ds(lo, ch)], out_sems[b]
            ).start()

        def wait_out(b):
            pltpu.make_async_copy(
                bufs[b], o_hbm.at[0, pl.ds(base, ch)], out_sems[b]
            ).wait()

        def process(t, b):
            wait_in(b)
            s, lo = chunk_coords(t)

            @pl.when(s == it)
            def _():
                for h in range(ch // ndh):
                    pltpu.sync_copy(nd_hbm.at[pl.ds(lo + h * ndh, ndh)], ndbuf)

                    def add_body(u, acc):
                        r = u // (cols // lanes)
                        jc = (u % (cols // lanes)) * lanes
                        br = h * ndh + r
                        bufs[b][br, pl.ds(jc, lanes)] = (
                            bufs[b][br, pl.ds(jc, lanes)]
                            + ndbuf[r, pl.ds(jc, lanes)] * _INV_STEPS
                        )
                        return acc

                    lax.fori_loop(0, groups, add_body, 0)

            start_out(t, b)

        start_in(0, 0)
        for b in range(_NBUF):
            if b >= 2:
                wait_out((b + 1) % _NBUF)
            start_in(b + 1, (b + 1) % _NBUF)
            process(b, b)

        ngroups = nt // _NBUF
        def group_body(g, carry):
            for b in range(_NBUF):
                t = g * _NBUF + b
                wait_out((b + 1) % _NBUF)
                start_in(t + 1, (b + 1) % _NBUF)
                process(t, b)
            return carry

        lax.fori_loop(1, ngroups, group_body, 0)

        for t in range(ngroups * _NBUF, nt):
            b = t % _NBUF
            wait_out((b + 1) % _NBUF)
            if t + 1 < nt:
                start_in(t + 1, (b + 1) % _NBUF)
            process(t, b)
        for t in range(nt - (_NBUF - 1), nt):
            wait_out(t % _NBUF)

    return k


def kernel(data, new_data, i):
    steps, rows, cols = data.shape
    iv = jnp.full((16,), jnp.asarray(i, jnp.int32))
    return _sc_kernel(steps, rows, cols)(iv, data, new_data)
